# baseline (speedup 1.0000x reference)
import functools

import jax
import jax.numpy as jnp
from jax import lax
from jax.experimental import pallas as pl
from jax.experimental.pallas import tpu as pltpu
from jax.experimental.pallas import tpu_sc as plsc

B = 16384
D_IN = 128
D_P = 32
D_OUT = 64
V = 1000000

NC, NS = 2, 16
NW = NC * NS
L = 16

SPAN = 32768
CC = 1024
NCH = SPAN // CC
CLAMP0 = 999040
STAGE = 128

_sc_mesh = plsc.VectorSubcoreMesh(core_axis_name="c", subcore_axis_name="s")


@functools.partial(
    pl.kernel,
    out_type=jax.ShapeDtypeStruct((B, 128), jnp.float32),
    mesh=_sc_mesh,
    scratch_types=[
        pltpu.VMEM((B,), jnp.int32),
        pltpu.VMEM((B,), jnp.int32),
        pltpu.VMEM((D_P, CC), jnp.float32),
        pltpu.VMEM((D_P, CC), jnp.float32),
        pltpu.VMEM((STAGE, 128), jnp.float32),
        pltpu.VMEM((STAGE,), jnp.int32),
        pltpu.SemaphoreType.DMA,
        pltpu.SemaphoreType.DMA,
        pltpu.SemaphoreType.DMA,
    ],
    compiler_params=pltpu.CompilerParams(needs_layout_passes=False),
)
def _gather_rows(tableT, idx_hbm, out_hbm, u_v, match_v, chunk0, chunk1,
                 stage_v, pos_v, sem_i, sem0, sem1):
    wid = lax.axis_index("s") * NC + lax.axis_index("c")
    lo = wid * SPAN

    idx_cp = pltpu.async_copy(idx_hbm, u_v, sem_i)

    iota = lax.iota(jnp.int32, L)
    neg1 = jnp.full((L,), -1, jnp.int32)
    for u in range(STAGE // L):
        pos_v[pl.ds(u * L, L)] = neg1

    zeros = jnp.zeros((L,), jnp.float32)

    def zero_body(r, carry):
        for u in range((128 - D_P) // L):
            stage_v[r, pl.ds(D_P + u * L, L)] = zeros
        return carry

    def start_full(c, buf, sem):
        w0 = pl.multiple_of(jnp.minimum(lo + c * CC, CLAMP0), 128)
        pltpu.async_copy(tableT.at[:, pl.ds(w0, CC)], buf, sem)

    start_full(0, chunk0, sem0)
    start_full(1, chunk1, sem1)

    lax.fori_loop(0, STAGE, zero_body, 0)
    idx_cp.wait()

    def scan_body(g4, cnt):
        for u in range(8):
            g = g4 * 8 + u
            rv = u_v[pl.ds(g * L, L)]
            bv = iota + g * L
            m = (rv >> 15) == wid
            j = rv & (SPAN - 1)
            plsc.store_compressed(
                match_v.at[pl.ds(cnt, L)], (j << 14) | bv, mask=m
            )
            cnt = cnt + jnp.sum(jnp.where(m, 1, 0))
        return cnt

    cnt = lax.fori_loop(0, B // L // 8, scan_body, 0)
    ngrp = (cnt + L - 1) // L

    def flush(fill):
        pltpu.async_copy(
            stage_v, out_hbm.at[plsc.Indices(pos_v, ignored_value=-1)], sem_i
        ).wait()
        for u in range(STAGE // L):
            pos_v[pl.ds(u * L, L)] = neg1
        return 0

    def wait_full(buf, sem):
        pltpu.make_async_copy(tableT.at[:, pl.ds(0, CC)], buf, sem).wait()

    def process(buf, c, fill):
        jbase = jnp.minimum(c * CC, CLAMP0 - lo)

        def rescan_body(g, cnt2):
            pv = match_v[pl.ds(g * L, L)]
            valid = (iota + g * L) < cnt
            jj = (pv >> 14) - jbase
            m = valid & (jj >= 0) & (jj < CC)
            plsc.store_compressed(
                u_v.at[pl.ds(cnt2, L)], (jj << 14) | (pv & 16383), mask=m
            )
            return cnt2 + jnp.sum(jnp.where(m, 1, 0))

        cnt2 = lax.fori_loop(0, ngrp, rescan_body, 0)

        def ext_body(h, fill):
            fill = lax.cond(fill + L > STAGE, flush, lambda f: f, fill)
            pv = u_v[pl.ds(h * L, L)]
            m = (iota + h * L) < cnt2
            jj = pv >> 14
            bv = pv & 16383
            rows = fill + iota
            for c in range(D_P):
                cvec = jnp.full((L,), c, jnp.int32)
                v = plsc.load_gather(buf, [cvec, jj], mask=m)
                plsc.store_scatter(stage_v, [rows, cvec], v, mask=m)
            plsc.store_scatter(pos_v, [rows], bv, mask=m)
            return fill + jnp.sum(jnp.where(m, 1, 0))

        return lax.fori_loop(0, (cnt2 + L - 1) // L, ext_body, fill)

    def two_chunks(g, fill):
        c0 = 2 * g
        wait_full(chunk0, sem0)
        fill = process(chunk0, c0, fill)

        @pl.when(g < NCH // 2 - 1)
        def _():
            start_full(c0 + 2, chunk0, sem0)

        wait_full(chunk1, sem1)
        fill = process(chunk1, c0 + 1, fill)

        @pl.when(g < NCH // 2 - 1)
        def _():
            start_full(c0 + 3, chunk1, sem1)

        return fill

    fill = lax.fori_loop(0, NCH // 2, two_chunks, 0)
    flush(fill)


def _ff_body(x_ref, e_ref, wt_ref, wpt_ref, bias_ref, out_ref):
    acc = jnp.dot(x_ref[...], wt_ref[...], preferred_element_type=jnp.float32)
    acc += jnp.dot(e_ref[...], wpt_ref[...], preferred_element_type=jnp.float32)
    out_ref[...] = acc + bias_ref[...]


BLK = 4096


def kernel(inputs, prompt, prompt_table, W_prompt, b_prompt, W, b):
    idx = prompt.astype(jnp.int32)
    embed = _gather_rows(prompt_table.T, idx)

    wt = W.T
    wpt = jnp.zeros((128, D_OUT), jnp.float32).at[:D_P].set(W_prompt.T)
    bias = (b + b_prompt).reshape(1, D_OUT)

    grid = (B // BLK,)
    out = pl.pallas_call(
        _ff_body,
        grid=grid,
        in_specs=[
            pl.BlockSpec((BLK, D_IN), lambda i: (i, 0)),
            pl.BlockSpec((BLK, 128), lambda i: (i, 0)),
            pl.BlockSpec((D_IN, D_OUT), lambda i: (0, 0)),
            pl.BlockSpec((128, D_OUT), lambda i: (0, 0)),
            pl.BlockSpec((1, D_OUT), lambda i: (0, 0)),
        ],
        out_specs=pl.BlockSpec((BLK, D_OUT), lambda i: (i, 0)),
        out_shape=jax.ShapeDtypeStruct((B, D_OUT), jnp.float32),
    )(inputs, embed, wt, wpt, bias)
    return out
